# baseline (speedup 1.0000x reference)
import jax
import jax.numpy as jnp
from jax import lax
from jax.experimental import pallas as pl
from jax.experimental.pallas import tpu as pltpu
from jax.experimental.pallas import tpu_sc as plsc

_N = 10000
_E = 320000
_D = 128
_DH = _D // 2
_NC = 2
_NS = 16
_NW = _NC * _NS
_CK = 80
_SUB = 640
_NBUF = 8

_mesh = plsc.VectorSubcoreMesh(
    core_axis_name="c", subcore_axis_name="s", num_cores=_NC, num_subcores=_NS
)


def _deg_body(src_hbm, dst_hbm, z_hbm, o_hbm, out_o_hbm, out_i_hbm,
              srcv, dstv, ones_v, zb, deg_o_sp, deg_i_sp):
    c = lax.axis_index("c")
    s = lax.axis_index("s")
    wid = s * _NC + c
    base = s * _SUB
    nblk = jnp.where(s < _NS - 1, _SUB // _CK, (_N - (_NS - 1) * _SUB) // _CK)
    nch = (_E // _CK) // _NW

    pltpu.sync_copy(z_hbm, zb)
    pltpu.sync_copy(o_hbm, ones_v)

    def init_loop(k, carry):
        pltpu.sync_copy(zb, deg_o_sp.at[pl.ds(base + k * _CK, _CK)])
        pltpu.sync_copy(zb, deg_i_sp.at[pl.ds(base + k * _CK, _CK)])
        return carry

    lax.fori_loop(0, nblk, init_loop, 0)

    pltpu.sync_copy(src_hbm.at[wid], srcv)
    pltpu.sync_copy(dst_hbm.at[wid], dstv)
    plsc.subcore_barrier()

    def body(ci, carry):
        pltpu.sync_copy(ones_v, deg_o_sp.at[srcv.at[ci]], add=True)
        pltpu.sync_copy(ones_v, deg_i_sp.at[dstv.at[ci]], add=True)
        return carry

    lax.fori_loop(0, nch, body, 0)
    plsc.subcore_barrier()

    def rb_loop(k, carry):
        pltpu.sync_copy(deg_o_sp.at[pl.ds(base + k * _CK, _CK)], zb)
        pltpu.sync_copy(zb, out_o_hbm.at[c, pl.ds(base + k * _CK, _CK)])
        pltpu.sync_copy(deg_i_sp.at[pl.ds(base + k * _CK, _CK)], zb)
        pltpu.sync_copy(zb, out_i_hbm.at[c, pl.ds(base + k * _CK, _CK)])
        return carry

    lax.fori_loop(0, nblk, rb_loop, 0)


_deg_call = pl.kernel(
    _deg_body,
    out_type=(
        jax.ShapeDtypeStruct((_NC, _N, 8), jnp.float32),
        jax.ShapeDtypeStruct((_NC, _N, 8), jnp.float32),
    ),
    mesh=_mesh,
    compiler_params=pltpu.CompilerParams(use_tc_tiling_on_sc=False),
    scratch_types=[
        pltpu.VMEM(((_E // _CK) // _NW, _CK), jnp.int32),
        pltpu.VMEM(((_E // _CK) // _NW, _CK), jnp.int32),
        pltpu.VMEM((_CK, 8), jnp.float32),
        pltpu.VMEM((_CK, 8), jnp.float32),
        pltpu.VMEM_SHARED((_N, 8), jnp.float32),
        pltpu.VMEM_SHARED((_N, 8), jnp.float32),
    ],
)


def _make_agg(num_edges):
    nch = (num_edges // _CK) // _NS

    def body(xn_l_hbm, xn_r_hbm, gidx_hbm, sidx_hbm, out_hbm,
             gv, sv, *rest):
        bufs = rest[:_NBUF]
        zb = rest[_NBUF]
        acc_sp = rest[_NBUF + 1]
        sems = rest[_NBUF + 2:]
        c = lax.axis_index("c")
        s = lax.axis_index("s")
        base = s * _SUB
        nblk = jnp.where(s < _NS - 1, _SUB // _CK,
                         (_N - (_NS - 1) * _SUB) // _CK)

        zeros16 = jnp.zeros((16,), jnp.float32)

        def zloop(i, carry):
            zb[i // (_DH // 16), pl.ds((i % (_DH // 16)) * 16, 16)] = zeros16
            return carry

        lax.fori_loop(0, _CK * (_DH // 16), zloop, 0)

        def init_loop(k, carry):
            pltpu.sync_copy(zb, acc_sp.at[pl.ds(base + k * _CK, _CK)])
            return carry

        lax.fori_loop(0, nblk, init_loop, 0)

        pltpu.sync_copy(gidx_hbm.at[s], gv)
        pltpu.sync_copy(sidx_hbm.at[s], sv)
        plsc.subcore_barrier()

        nbuf = _NBUF

        def run(tab):
            for j in range(nbuf):
                pltpu.async_copy(tab.at[gv.at[j]], bufs[j], sems[j])

            def body_i(i, carry):
                for j in range(nbuf):
                    cj = nbuf * i + j
                    pltpu.make_async_copy(
                        tab.at[gv.at[cj]], bufs[j], sems[j]).wait()
                    pltpu.sync_copy(bufs[j], acc_sp.at[sv.at[cj]], add=True)

                    @pl.when(cj + nbuf < nch)
                    def _():
                        pltpu.async_copy(
                            tab.at[gv.at[cj + nbuf]], bufs[j], sems[j])

                return carry

            lax.fori_loop(0, nch // nbuf, body_i, 0)
            for j in range(nch % nbuf):
                cj = (nch // nbuf) * nbuf + j
                pltpu.make_async_copy(
                    tab.at[gv.at[cj]], bufs[j], sems[j]).wait()
                pltpu.sync_copy(bufs[j], acc_sp.at[sv.at[cj]], add=True)

        @pl.when(c == 0)
        def _():
            run(xn_l_hbm)

        @pl.when(c == 1)
        def _():
            run(xn_r_hbm)

        plsc.subcore_barrier()

        def rb_loop(k, carry):
            pltpu.sync_copy(acc_sp.at[pl.ds(base + k * _CK, _CK)], zb)
            pltpu.sync_copy(zb, out_hbm.at[c, pl.ds(base + k * _CK, _CK)])
            return carry

        lax.fori_loop(0, nblk, rb_loop, 0)

    return pl.kernel(
        body,
        out_type=jax.ShapeDtypeStruct((_NC, _N, _DH), jnp.float32),
        mesh=_mesh,
        compiler_params=pltpu.CompilerParams(use_tc_tiling_on_sc=False),
        scratch_types=[
            pltpu.VMEM((nch, _CK), jnp.int32),
            pltpu.VMEM((nch, _CK), jnp.int32),
        ] + [pltpu.VMEM((_CK, _DH), jnp.float32) for _ in range(_NBUF)] + [
            pltpu.VMEM((_CK, _DH), jnp.float32),
            pltpu.VMEM_SHARED((_N, _DH), jnp.float32),
        ] + [pltpu.SemaphoreType.DMA for _ in range(_NBUF)],
    )


_agg_e = _make_agg(_E)

_BR = 1000


def _scale_body(x_ref, dp_ref, ol_ref, or_ref):
    d = dp_ref[0] + dp_ref[1]
    xn = x_ref[...] * (1.0 / jnp.maximum(d, 1.0))
    ol_ref[...] = xn[:, :_DH]
    or_ref[...] = xn[:, _DH:]


_scale = pl.pallas_call(
    _scale_body,
    grid=(_N // _BR,),
    in_specs=[
        pl.BlockSpec((_BR, _D), lambda i: (i, 0)),
        pl.BlockSpec((2, _BR, 1), lambda i: (0, i, 0)),
    ],
    out_specs=[
        pl.BlockSpec((_BR, _DH), lambda i: (i, 0)),
        pl.BlockSpec((_BR, _DH), lambda i: (i, 0)),
    ],
    out_shape=[
        jax.ShapeDtypeStruct((_N, _DH), jnp.float32),
        jax.ShapeDtypeStruct((_N, _DH), jnp.float32),
    ],
)


def _dense_hn_body(x_ref, p_ref, wt_ref, wbt_ref, wbb_ref, b_ref, dp_ref,
                   h_ref, hnl_ref, hnr_ref):
    h = jnp.dot(x_ref[...], wt_ref[...], preferred_element_type=jnp.float32)
    h = h + jnp.dot(p_ref[0], wbt_ref[...], preferred_element_type=jnp.float32)
    h = h + jnp.dot(p_ref[1], wbb_ref[...], preferred_element_type=jnp.float32)
    h = h + b_ref[...]
    h = jnp.maximum(h, 0.0)
    h_ref[...] = h
    d = jnp.sum(dp_ref[...], axis=0)
    hn = h * (1.0 / jnp.maximum(d, 1.0))
    hnl_ref[...] = hn[:, :_DH]
    hnr_ref[...] = hn[:, _DH:]


def _make_dense_hn(num_deg_parts):
    return pl.pallas_call(
        _dense_hn_body,
        grid=(_N // _BR,),
        in_specs=[
            pl.BlockSpec((_BR, _D), lambda i: (i, 0)),
            pl.BlockSpec((2, _BR, _DH), lambda i: (0, i, 0)),
            pl.BlockSpec((_D, _D), lambda i: (0, 0)),
            pl.BlockSpec((_DH, _D), lambda i: (0, 0)),
            pl.BlockSpec((_DH, _D), lambda i: (0, 0)),
            pl.BlockSpec((1, _D), lambda i: (0, 0)),
            pl.BlockSpec((num_deg_parts, _BR, 1), lambda i: (0, i, 0)),
        ],
        out_specs=[
            pl.BlockSpec((_BR, _D), lambda i: (i, 0)),
            pl.BlockSpec((_BR, _DH), lambda i: (i, 0)),
            pl.BlockSpec((_BR, _DH), lambda i: (i, 0)),
        ],
        out_shape=[
            jax.ShapeDtypeStruct((_N, _D), jnp.float32),
            jax.ShapeDtypeStruct((_N, _DH), jnp.float32),
            jax.ShapeDtypeStruct((_N, _DH), jnp.float32),
        ],
    )


_dense_hn2 = _make_dense_hn(2)
_dense_hn4 = _make_dense_hn(4)


def _dense_last_body(x_ref, pa_ref, pb_ref, wt_ref, wbt_ref, wbb_ref, b_ref,
                     h_ref):
    h = jnp.dot(x_ref[...], wt_ref[...], preferred_element_type=jnp.float32)
    h = h + jnp.dot(pa_ref[0] + pb_ref[0], wbt_ref[...],
                    preferred_element_type=jnp.float32)
    h = h + jnp.dot(pa_ref[1] + pb_ref[1], wbb_ref[...],
                    preferred_element_type=jnp.float32)
    h_ref[...] = h + b_ref[...]


_dense_last = pl.pallas_call(
    _dense_last_body,
    grid=(_N // _BR,),
    in_specs=[
        pl.BlockSpec((_BR, _D), lambda i: (i, 0)),
        pl.BlockSpec((2, _BR, _DH), lambda i: (0, i, 0)),
        pl.BlockSpec((2, _BR, _DH), lambda i: (0, i, 0)),
        pl.BlockSpec((_D, _D), lambda i: (0, 0)),
        pl.BlockSpec((_DH, _D), lambda i: (0, 0)),
        pl.BlockSpec((_DH, _D), lambda i: (0, 0)),
        pl.BlockSpec((1, _D), lambda i: (0, 0)),
    ],
    out_specs=pl.BlockSpec((_BR, _D), lambda i: (i, 0)),
    out_shape=jax.ShapeDtypeStruct((_N, _D), jnp.float32),
)


def kernel(x, edge_index, W1, b1, W2, b2, W3, b3):
    src = edge_index[0]
    dst = edge_index[1]
    nchw = (_E // _CK) // _NW
    nchs = (_E // _CK) // _NS
    src_w = src.reshape(_NW, nchw, _CK)
    dst_w = dst.reshape(_NW, nchw, _CK)
    src_s = src.reshape(_NS, nchs, _CK)
    dst_s = dst.reshape(_NS, nchs, _CK)

    zrow = jnp.zeros((_CK, 8), jnp.float32)
    orow = jnp.ones((_CK, 8), jnp.float32)
    out_o, out_i = _deg_call(src_w, dst_w, zrow, orow)
    d_o = out_o[:, :, 0:1]
    d_i = out_i[:, :, 0:1]

    xn0_l, xn0_r = _scale(x, d_o)
    p0 = _agg_e(xn0_l, xn0_r, src_s, dst_s)
    h1, h1n_l, h1n_r = _dense_hn2(
        x, p0, W1[:_D], W1[_D:_D + _DH], W1[_D + _DH:], b1.reshape(1, _D), d_i)

    p1 = _agg_e(h1n_l, h1n_r, dst_s, src_s)
    d_u = jnp.concatenate([d_o, d_i], axis=0)
    h2, h2n_l, h2n_r = _dense_hn4(
        h1, p1, W2[:_D], W2[_D:_D + _DH], W2[_D + _DH:], b2.reshape(1, _D), d_u)

    p2a = _agg_e(h2n_l, h2n_r, src_s, dst_s)
    p2b = _agg_e(h2n_l, h2n_r, dst_s, src_s)
    h3 = _dense_last(
        h2, p2a, p2b, W3[:_D], W3[_D:_D + _DH], W3[_D + _DH:],
        b3.reshape(1, _D))
    return h3


# speedup vs baseline: 1.0707x; 1.0707x over previous
import jax
import jax.numpy as jnp
from jax import lax
from jax.experimental import pallas as pl
from jax.experimental.pallas import tpu as pltpu
from jax.experimental.pallas import tpu_sc as plsc

_N = 10000
_E = 320000
_D = 128
_DH = _D // 2
_NC = 2
_NS = 16
_NW = _NC * _NS
_CK = 80
_SUB = 640
_NBUF = 8

_mesh = plsc.VectorSubcoreMesh(
    core_axis_name="c", subcore_axis_name="s", num_cores=_NC, num_subcores=_NS
)


def _deg_body(src_hbm, dst_hbm, z_hbm, o_hbm, out_o_hbm, out_i_hbm,
              srcv, dstv, ones_v, zb, deg_o_sp, deg_i_sp, sem):
    c = lax.axis_index("c")
    s = lax.axis_index("s")
    wid = s * _NC + c
    base = s * _SUB
    nblk = jnp.where(s < _NS - 1, _SUB // _CK, (_N - (_NS - 1) * _SUB) // _CK)
    nch = (_E // _CK) // _NW

    pltpu.sync_copy(z_hbm, zb)
    pltpu.sync_copy(o_hbm, ones_v)

    def init_loop(k, carry):
        pltpu.sync_copy(zb, deg_o_sp.at[pl.ds(base + k * _CK, _CK)])
        pltpu.sync_copy(zb, deg_i_sp.at[pl.ds(base + k * _CK, _CK)])
        return carry

    lax.fori_loop(0, nblk, init_loop, 0)

    pltpu.sync_copy(src_hbm.at[wid], srcv)
    pltpu.sync_copy(dst_hbm.at[wid], dstv)
    plsc.subcore_barrier()

    def body(w, carry):
        for j in range(5):
            ci = 5 * w + j
            pltpu.async_copy(ones_v, deg_o_sp.at[srcv.at[ci]], sem, add=True)
            pltpu.async_copy(ones_v, deg_i_sp.at[dstv.at[ci]], sem, add=True)
        for j in range(5):
            ci = 5 * w + j
            pltpu.make_async_copy(
                ones_v, deg_o_sp.at[srcv.at[ci]], sem).wait()
            pltpu.make_async_copy(
                ones_v, deg_i_sp.at[dstv.at[ci]], sem).wait()
        return carry

    lax.fori_loop(0, nch // 5, body, 0)
    plsc.subcore_barrier()

    def rb_loop(k, carry):
        pltpu.sync_copy(deg_o_sp.at[pl.ds(base + k * _CK, _CK)], zb)
        pltpu.sync_copy(zb, out_o_hbm.at[c, pl.ds(base + k * _CK, _CK)])
        pltpu.sync_copy(deg_i_sp.at[pl.ds(base + k * _CK, _CK)], zb)
        pltpu.sync_copy(zb, out_i_hbm.at[c, pl.ds(base + k * _CK, _CK)])
        return carry

    lax.fori_loop(0, nblk, rb_loop, 0)


_deg_call = pl.kernel(
    _deg_body,
    out_type=(
        jax.ShapeDtypeStruct((_NC, _N, 8), jnp.float32),
        jax.ShapeDtypeStruct((_NC, _N, 8), jnp.float32),
    ),
    mesh=_mesh,
    compiler_params=pltpu.CompilerParams(use_tc_tiling_on_sc=False),
    scratch_types=[
        pltpu.VMEM(((_E // _CK) // _NW, _CK), jnp.int32),
        pltpu.VMEM(((_E // _CK) // _NW, _CK), jnp.int32),
        pltpu.VMEM((_CK, 8), jnp.float32),
        pltpu.VMEM((_CK, 8), jnp.float32),
        pltpu.VMEM_SHARED((_N, 8), jnp.float32),
        pltpu.VMEM_SHARED((_N, 8), jnp.float32),
        pltpu.SemaphoreType.DMA,
    ],
)


def _make_agg(num_edges, undirected=False):
    nch = (num_edges // _CK) // _NS

    def body(xn_l_hbm, xn_r_hbm, gidx_hbm, sidx_hbm, out_hbm,
             gv, sv, *rest):
        bufs = rest[:_NBUF]
        zb = rest[_NBUF]
        acc_sp = rest[_NBUF + 1]
        sems = rest[_NBUF + 2:]
        c = lax.axis_index("c")
        s = lax.axis_index("s")
        base = s * _SUB
        nblk = jnp.where(s < _NS - 1, _SUB // _CK,
                         (_N - (_NS - 1) * _SUB) // _CK)

        zeros16 = jnp.zeros((16,), jnp.float32)

        def zloop(i, carry):
            zb[i // (_DH // 16), pl.ds((i % (_DH // 16)) * 16, 16)] = zeros16
            return carry

        lax.fori_loop(0, _CK * (_DH // 16), zloop, 0)

        def init_loop(k, carry):
            pltpu.sync_copy(zb, acc_sp.at[pl.ds(base + k * _CK, _CK)])
            return carry

        lax.fori_loop(0, nblk, init_loop, 0)

        pltpu.sync_copy(gidx_hbm.at[s], gv)
        pltpu.sync_copy(sidx_hbm.at[s], sv)
        plsc.subcore_barrier()

        nbuf = _NBUF

        def run(tab, gvx, svx):
            for j in range(nbuf):
                pltpu.async_copy(tab.at[gvx.at[j]], bufs[j], sems[j])

            def body_i(i, carry):
                for j in range(nbuf):
                    cj = nbuf * i + j
                    pltpu.make_async_copy(
                        tab.at[gvx.at[cj]], bufs[j], sems[j]).wait()
                    pltpu.sync_copy(bufs[j], acc_sp.at[svx.at[cj]], add=True)

                    @pl.when(cj + nbuf < nch)
                    def _():
                        pltpu.async_copy(
                            tab.at[gvx.at[cj + nbuf]], bufs[j], sems[j])

                return carry

            lax.fori_loop(0, nch // nbuf, body_i, 0)
            for j in range(nch % nbuf):
                cj = (nch // nbuf) * nbuf + j
                pltpu.make_async_copy(
                    tab.at[gvx.at[cj]], bufs[j], sems[j]).wait()
                pltpu.sync_copy(bufs[j], acc_sp.at[svx.at[cj]], add=True)

        @pl.when(c == 0)
        def _():
            run(xn_l_hbm, gv, sv)
            if undirected:
                run(xn_l_hbm, sv, gv)

        @pl.when(c == 1)
        def _():
            run(xn_r_hbm, gv, sv)
            if undirected:
                run(xn_r_hbm, sv, gv)

        plsc.subcore_barrier()

        for k in range(_SUB // _CK):
            def _rb1(k=k):
                pltpu.async_copy(
                    acc_sp.at[pl.ds(base + k * _CK, _CK)], bufs[k], sems[k])
            pl.when(k < nblk)(_rb1)
        for k in range(_SUB // _CK):
            def _rb2(k=k):
                pltpu.make_async_copy(
                    acc_sp.at[pl.ds(base + k * _CK, _CK)], bufs[k],
                    sems[k]).wait()
                pltpu.async_copy(
                    bufs[k], out_hbm.at[c, pl.ds(base + k * _CK, _CK)],
                    sems[k])
            pl.when(k < nblk)(_rb2)
        for k in range(_SUB // _CK):
            def _rb3(k=k):
                pltpu.make_async_copy(
                    bufs[k], out_hbm.at[c, pl.ds(base + k * _CK, _CK)],
                    sems[k]).wait()
            pl.when(k < nblk)(_rb3)

    return pl.kernel(
        body,
        out_type=jax.ShapeDtypeStruct((_NC, _N, _DH), jnp.float32),
        mesh=_mesh,
        compiler_params=pltpu.CompilerParams(use_tc_tiling_on_sc=False),
        scratch_types=[
            pltpu.VMEM((nch, _CK), jnp.int32),
            pltpu.VMEM((nch, _CK), jnp.int32),
        ] + [pltpu.VMEM((_CK, _DH), jnp.float32) for _ in range(_NBUF)] + [
            pltpu.VMEM((_CK, _DH), jnp.float32),
            pltpu.VMEM_SHARED((_N, _DH), jnp.float32),
        ] + [pltpu.SemaphoreType.DMA for _ in range(_NBUF)],
    )


_agg_e = _make_agg(_E)
_agg_u = _make_agg(_E, undirected=True)

_BR = 1000


def _scale_body(x_ref, dp_ref, ol_ref, or_ref):
    d = dp_ref[0] + dp_ref[1]
    xn = x_ref[...] * (1.0 / jnp.maximum(d, 1.0))
    ol_ref[...] = xn[:, :_DH]
    or_ref[...] = xn[:, _DH:]


_scale = pl.pallas_call(
    _scale_body,
    grid=(_N // _BR,),
    in_specs=[
        pl.BlockSpec((_BR, _D), lambda i: (i, 0)),
        pl.BlockSpec((2, _BR, 1), lambda i: (0, i, 0)),
    ],
    out_specs=[
        pl.BlockSpec((_BR, _DH), lambda i: (i, 0)),
        pl.BlockSpec((_BR, _DH), lambda i: (i, 0)),
    ],
    out_shape=[
        jax.ShapeDtypeStruct((_N, _DH), jnp.float32),
        jax.ShapeDtypeStruct((_N, _DH), jnp.float32),
    ],
)


def _dense_hn_body(x_ref, p_ref, wt_ref, wbt_ref, wbb_ref, b_ref, dp_ref,
                   h_ref, hnl_ref, hnr_ref):
    h = jnp.dot(x_ref[...], wt_ref[...], preferred_element_type=jnp.float32)
    h = h + jnp.dot(p_ref[0], wbt_ref[...], preferred_element_type=jnp.float32)
    h = h + jnp.dot(p_ref[1], wbb_ref[...], preferred_element_type=jnp.float32)
    h = h + b_ref[...]
    h = jnp.maximum(h, 0.0)
    h_ref[...] = h
    d = jnp.sum(dp_ref[...], axis=0)
    hn = h * (1.0 / jnp.maximum(d, 1.0))
    hnl_ref[...] = hn[:, :_DH]
    hnr_ref[...] = hn[:, _DH:]


def _make_dense_hn(num_deg_parts):
    return pl.pallas_call(
        _dense_hn_body,
        grid=(_N // _BR,),
        in_specs=[
            pl.BlockSpec((_BR, _D), lambda i: (i, 0)),
            pl.BlockSpec((2, _BR, _DH), lambda i: (0, i, 0)),
            pl.BlockSpec((_D, _D), lambda i: (0, 0)),
            pl.BlockSpec((_DH, _D), lambda i: (0, 0)),
            pl.BlockSpec((_DH, _D), lambda i: (0, 0)),
            pl.BlockSpec((1, _D), lambda i: (0, 0)),
            pl.BlockSpec((num_deg_parts, _BR, 1), lambda i: (0, i, 0)),
        ],
        out_specs=[
            pl.BlockSpec((_BR, _D), lambda i: (i, 0)),
            pl.BlockSpec((_BR, _DH), lambda i: (i, 0)),
            pl.BlockSpec((_BR, _DH), lambda i: (i, 0)),
        ],
        out_shape=[
            jax.ShapeDtypeStruct((_N, _D), jnp.float32),
            jax.ShapeDtypeStruct((_N, _DH), jnp.float32),
            jax.ShapeDtypeStruct((_N, _DH), jnp.float32),
        ],
    )


_dense_hn2 = _make_dense_hn(2)
_dense_hn4 = _make_dense_hn(4)


def _dense_last_body(x_ref, p_ref, wt_ref, wbt_ref, wbb_ref, b_ref, h_ref):
    h = jnp.dot(x_ref[...], wt_ref[...], preferred_element_type=jnp.float32)
    h = h + jnp.dot(p_ref[0], wbt_ref[...], preferred_element_type=jnp.float32)
    h = h + jnp.dot(p_ref[1], wbb_ref[...], preferred_element_type=jnp.float32)
    h_ref[...] = h + b_ref[...]


_dense_last = pl.pallas_call(
    _dense_last_body,
    grid=(_N // _BR,),
    in_specs=[
        pl.BlockSpec((_BR, _D), lambda i: (i, 0)),
        pl.BlockSpec((2, _BR, _DH), lambda i: (0, i, 0)),
        pl.BlockSpec((_D, _D), lambda i: (0, 0)),
        pl.BlockSpec((_DH, _D), lambda i: (0, 0)),
        pl.BlockSpec((_DH, _D), lambda i: (0, 0)),
        pl.BlockSpec((1, _D), lambda i: (0, 0)),
    ],
    out_specs=pl.BlockSpec((_BR, _D), lambda i: (i, 0)),
    out_shape=jax.ShapeDtypeStruct((_N, _D), jnp.float32),
)


def kernel(x, edge_index, W1, b1, W2, b2, W3, b3):
    src = edge_index[0]
    dst = edge_index[1]
    nchw = (_E // _CK) // _NW
    nchs = (_E // _CK) // _NS
    src_w = src.reshape(_NW, nchw, _CK)
    dst_w = dst.reshape(_NW, nchw, _CK)
    src_s = src.reshape(_NS, nchs, _CK)
    dst_s = dst.reshape(_NS, nchs, _CK)

    zrow = jnp.zeros((_CK, 8), jnp.float32)
    orow = jnp.ones((_CK, 8), jnp.float32)
    out_o, out_i = _deg_call(src_w, dst_w, zrow, orow)
    d_o = out_o[:, :, 0:1]
    d_i = out_i[:, :, 0:1]

    xn0_l, xn0_r = _scale(x, d_o)
    p0 = _agg_e(xn0_l, xn0_r, src_s, dst_s)
    h1, h1n_l, h1n_r = _dense_hn2(
        x, p0, W1[:_D], W1[_D:_D + _DH], W1[_D + _DH:], b1.reshape(1, _D), d_i)

    p1 = _agg_e(h1n_l, h1n_r, dst_s, src_s)
    d_u = jnp.concatenate([d_o, d_i], axis=0)
    h2, h2n_l, h2n_r = _dense_hn4(
        h1, p1, W2[:_D], W2[_D:_D + _DH], W2[_D + _DH:], b2.reshape(1, _D), d_u)

    p2 = _agg_u(h2n_l, h2n_r, src_s, dst_s)
    h3 = _dense_last(
        h2, p2, W3[:_D], W3[_D:_D + _DH], W3[_D + _DH:], b3.reshape(1, _D))
    return h3
